# baseline (speedup 1.0000x reference)
import functools

import jax
import jax.numpy as jnp
from jax import lax
from jax.experimental import pallas as pl
from jax.experimental.pallas import tpu as pltpu
from jax.experimental.pallas import tpu_sc as plsc

N = 10000
E = 320000
D = 150
DP = 160
PASSES = 4
NC = 2
NS = 16
NW = NC * NS
EPW = E // NW
K = 80
CH = EPW // K
G = 5
CHG = CH // G
NP = 10112
RPS = NP // NS
RB = 1000


def _build_sc_segment_sum():
    mesh = plsc.VectorSubcoreMesh(
        core_axis_name="c", subcore_axis_name="s", num_cores=NC, num_subcores=NS
    )

    @functools.partial(
        pl.kernel,
        out_type=jax.ShapeDtypeStruct((NC * NP, DP), jnp.float32),
        mesh=mesh,
        scratch_types=[
            pltpu.VMEM((CHG * K,), jnp.int32),
            pltpu.VMEM((CHG * K,), jnp.int32),
            pltpu.VMEM((K, DP), jnp.float32),
            pltpu.VMEM((K, DP), jnp.float32),
            pltpu.VMEM_SHARED((NP, DP), jnp.float32),
            pltpu.SemaphoreType.DMA,
            pltpu.SemaphoreType.DMA,
        ],
        compiler_params=pltpu.CompilerParams(use_tc_tiling_on_sc=False),
    )
    def seg_sum(h_hbm, src_hbm, dst_hbm, zeros_hbm, out_hbm,
                src_v, dst_v, rows_a, rows_b, acc_sh, sem_a, sem_b):
        c = lax.axis_index("c")
        s = lax.axis_index("s")
        wid = s * NC + c
        pltpu.sync_copy(zeros_hbm, acc_sh.at[pl.ds(s * RPS, RPS)])
        plsc.subcore_barrier()

        def group(g, carry):
            base = wid * EPW + g * (CHG * K)
            pltpu.sync_copy(src_hbm.at[pl.ds(base, CHG * K)], src_v)
            pltpu.sync_copy(dst_hbm.at[pl.ds(base, CHG * K)], dst_v)
            pltpu.async_copy(h_hbm.at[src_v.at[pl.ds(0, K)]], rows_a, sem_a)
            pltpu.async_copy(h_hbm.at[src_v.at[pl.ds(K, K)]], rows_b, sem_b)

            def pair(jj, carry2):
                j0 = 2 * jj
                j1 = j0 + 1
                pltpu.make_async_copy(
                    h_hbm.at[src_v.at[pl.ds(j0 * K, K)]], rows_a, sem_a).wait()
                pltpu.sync_copy(rows_a, acc_sh.at[dst_v.at[pl.ds(j0 * K, K)]], add=True)
                pltpu.async_copy(h_hbm.at[src_v.at[pl.ds((j0 + 2) * K, K)]],
                                 rows_a, sem_a)

                pltpu.make_async_copy(
                    h_hbm.at[src_v.at[pl.ds(j1 * K, K)]], rows_b, sem_b).wait()
                pltpu.sync_copy(rows_b, acc_sh.at[dst_v.at[pl.ds(j1 * K, K)]], add=True)

                @pl.when(jj < CHG // 2 - 1)
                def _():
                    pltpu.async_copy(h_hbm.at[src_v.at[pl.ds((j1 + 2) * K, K)]],
                                     rows_b, sem_b)

                return carry2

            lax.fori_loop(0, CHG // 2, pair, 0)
            pltpu.make_async_copy(
                h_hbm.at[src_v.at[pl.ds((CHG - 1) * K, K)]], rows_a, sem_a).wait()
            pltpu.sync_copy(rows_a, acc_sh.at[dst_v.at[pl.ds((CHG - 1) * K, K)]], add=True)
            return carry

        lax.fori_loop(0, G, group, 0)
        plsc.subcore_barrier()
        pltpu.sync_copy(acc_sh.at[pl.ds(s * RPS, RPS)],
                        out_hbm.at[pl.ds(c * NP + s * RPS, RPS)])

    return seg_sum


def _gru_body(p_ref, h_ref, wih_ref, whh_ref, bih_ref, bhh_ref, out_ref):
    x = p_ref[0] + p_ref[1]
    h = h_ref[...]
    gi = jnp.dot(x, wih_ref[...], preferred_element_type=jnp.float32) + bih_ref[...]
    gh = jnp.dot(h, whh_ref[...], preferred_element_type=jnp.float32) + bhh_ref[...]
    i_r, i_z, i_n = gi[:, :DP], gi[:, DP:2 * DP], gi[:, 2 * DP:]
    h_r, h_z, h_n = gh[:, :DP], gh[:, DP:2 * DP], gh[:, 2 * DP:]
    r = jax.nn.sigmoid(i_r + h_r)
    z = jax.nn.sigmoid(i_z + h_z)
    n = jnp.tanh(i_n + r * h_n)
    out_ref[...] = (1.0 - z) * n + z * h


_tc_gru = pl.pallas_call(
    _gru_body,
    grid=(N // RB,),
    in_specs=[
        pl.BlockSpec((NC, RB, DP), lambda i: (0, i, 0)),
        pl.BlockSpec((RB, DP), lambda i: (i, 0)),
        pl.BlockSpec((DP, 3 * DP), lambda i: (0, 0)),
        pl.BlockSpec((DP, 3 * DP), lambda i: (0, 0)),
        pl.BlockSpec((1, 3 * DP), lambda i: (0, 0)),
        pl.BlockSpec((1, 3 * DP), lambda i: (0, 0)),
    ],
    out_specs=pl.BlockSpec((RB, DP), lambda i: (i, 0)),
    out_shape=jax.ShapeDtypeStruct((N, DP), jnp.float32),
)


RB2 = 632
GRID2 = NP // RB2


def _gru_last_body(p0_ref, p1_ref, h_ref, wih_ref, whh_ref, bih_ref, bhh_ref,
                   w1_ref, b1_ref, w2_ref, b2_ref, w3_ref, b3_ref,
                   out_ref, acc_ref):
    i = pl.program_id(0)
    x = p0_ref[...] + p1_ref[...]
    h = h_ref[...]
    gi = jnp.dot(x, wih_ref[...], preferred_element_type=jnp.float32) + bih_ref[...]
    gh = jnp.dot(h, whh_ref[...], preferred_element_type=jnp.float32) + bhh_ref[...]
    i_r, i_z, i_n = gi[:, :DP], gi[:, DP:2 * DP], gi[:, 2 * DP:]
    h_r, h_z, h_n = gh[:, :DP], gh[:, DP:2 * DP], gh[:, 2 * DP:]
    r = jax.nn.sigmoid(i_r + h_r)
    z = jax.nn.sigmoid(i_z + h_z)
    n = jnp.tanh(i_n + r * h_n)
    hn = (1.0 - z) * n + z * h
    rowid = jax.lax.broadcasted_iota(jnp.int32, (RB2, 1), 0) + i * RB2
    hn = jnp.where(rowid < N, hn, 0.0)

    @pl.when(i == 0)
    def _():
        acc_ref[...] = jnp.zeros((1, DP), jnp.float32)

    acc_ref[...] += jnp.sum(hn, axis=0, keepdims=True)

    @pl.when(i == GRID2 - 1)
    def _():
        g = acc_ref[...]
        g = jnp.log(g)
        g = jnp.where(jnp.isnan(g), 0.0, g)
        g = jnp.maximum(g, 0.0)
        y = jnp.dot(g, w1_ref[...], preferred_element_type=jnp.float32) + b1_ref[...]
        y = jnp.where(y >= 0.0, y, 0.01 * y)
        y = jnp.dot(y, w2_ref[...], preferred_element_type=jnp.float32) + b2_ref[...]
        y = jnp.where(y >= 0.0, y, 0.01 * y)
        y = jnp.dot(y, w3_ref[...], preferred_element_type=jnp.float32) + b3_ref[...]
        out_ref[...] = y


_tc_gru_last = pl.pallas_call(
    _gru_last_body,
    grid=(GRID2,),
    in_specs=[
        pl.BlockSpec((RB2, DP), lambda i: (i, 0)),
        pl.BlockSpec((RB2, DP), lambda i: (GRID2 + i, 0)),
        pl.BlockSpec((RB2, DP), lambda i: (i, 0)),
        pl.BlockSpec((DP, 3 * DP), lambda i: (0, 0)),
        pl.BlockSpec((DP, 3 * DP), lambda i: (0, 0)),
        pl.BlockSpec((1, 3 * DP), lambda i: (0, 0)),
        pl.BlockSpec((1, 3 * DP), lambda i: (0, 0)),
        pl.BlockSpec((DP, 80), lambda i: (0, 0)),
        pl.BlockSpec((1, 80), lambda i: (0, 0)),
        pl.BlockSpec((80, 80), lambda i: (0, 0)),
        pl.BlockSpec((1, 80), lambda i: (0, 0)),
        pl.BlockSpec((80, 16), lambda i: (0, 0)),
        pl.BlockSpec((1, 16), lambda i: (0, 0)),
    ],
    out_specs=pl.BlockSpec((1, 16), lambda i: (0, 0)),
    out_shape=jax.ShapeDtypeStruct((1, 16), jnp.float32),
    scratch_shapes=[pltpu.VMEM((1, DP), jnp.float32)],
)

_sc_segment_sum_cache = []


def _sc_segment_sum(h, src, dst, zeros):
    if not _sc_segment_sum_cache:
        _sc_segment_sum_cache.append(_build_sc_segment_sum())
    return _sc_segment_sum_cache[0](h, src, dst, zeros)


def _pad_gate_weights(w, b):
    w3 = w.reshape(3, D, D)
    wt = jnp.zeros((3, DP, DP), jnp.float32)
    wt = wt.at[:, :D, :D].set(jnp.transpose(w3, (0, 2, 1)))
    wt = jnp.transpose(wt, (1, 0, 2)).reshape(DP, 3 * DP)
    bp = jnp.zeros((3, DP), jnp.float32).at[:, :D].set(b.reshape(3, D)).reshape(1, 3 * DP)
    return wt, bp


def kernel(nodes, edge_index, W_ih, W_hh, b_ih, b_hh,
           fc1_w, fc1_b, fc2_w, fc2_b, fc3_w, fc3_b):
    src = edge_index[0].astype(jnp.int32)
    dst = edge_index[1].astype(jnp.int32)
    h = jnp.zeros((N, DP), jnp.float32).at[:, :D].set(nodes)
    zeros = jnp.zeros((RPS, DP), jnp.float32)

    wih, bih = _pad_gate_weights(W_ih, b_ih)
    whh, bhh = _pad_gate_weights(W_hh, b_hh)
    w1 = jnp.zeros((DP, 80), jnp.float32).at[:D, :].set(fc1_w.T)
    b1 = fc1_b.reshape(1, 80)
    w2 = fc2_w.T
    b2 = fc2_b.reshape(1, 80)
    w3 = jnp.zeros((80, 16), jnp.float32).at[:, :10].set(fc3_w.T)
    b3 = jnp.zeros((1, 16), jnp.float32).at[0, :10].set(fc3_b)

    for _ in range(PASSES - 1):
        partials = _sc_segment_sum(h, src, dst, zeros).reshape(NC, NP, DP)
        h = _tc_gru(partials, h, wih, whh, bih, bhh)

    pflat = _sc_segment_sum(h, src, dst, zeros)
    out = _tc_gru_last(pflat, pflat, h, wih, whh, bih, bhh,
                       w1, b1, w2, b2, w3, b3)
    return out[0, :10]


# speedup vs baseline: 1.0320x; 1.0320x over previous
import functools

import jax
import jax.numpy as jnp
from jax import lax
from jax.experimental import pallas as pl
from jax.experimental.pallas import tpu as pltpu
from jax.experimental.pallas import tpu_sc as plsc

N = 10000
E = 320000
D = 150
DP = 160
PASSES = 4
NC = 2
NS = 16
NW = NC * NS
EPW = E // NW
K = 80
CH = EPW // K
G = 5
CHG = CH // G
NP = 10112
RPS = NP // NS
RB = 1000


def _build_sc_segment_sum():
    mesh = plsc.VectorSubcoreMesh(
        core_axis_name="c", subcore_axis_name="s", num_cores=NC, num_subcores=NS
    )

    @functools.partial(
        pl.kernel,
        out_type=jax.ShapeDtypeStruct((NC * NP, DP), jnp.float32),
        mesh=mesh,
        scratch_types=[
            pltpu.VMEM((CHG * K,), jnp.int32),
            pltpu.VMEM((CHG * K,), jnp.int32),
            pltpu.VMEM((K, DP), jnp.float32),
            pltpu.VMEM((K, DP), jnp.float32),
            pltpu.VMEM_SHARED((NP, DP), jnp.float32),
            pltpu.SemaphoreType.DMA,
            pltpu.SemaphoreType.DMA,
        ],
        compiler_params=pltpu.CompilerParams(use_tc_tiling_on_sc=False),
    )
    def seg_sum(h_hbm, src_hbm, dst_hbm, out_hbm,
                src_v, dst_v, rows_a, rows_b, acc_sh, sem_a, sem_b):
        c = lax.axis_index("c")
        s = lax.axis_index("s")
        wid = s * NC + c
        z16 = jnp.zeros((16,), jnp.float32)

        def zrow(r, carry):
            def zcol(q, carry2):
                rows_a[r, pl.ds(q * 16, 16)] = z16
                return carry2

            lax.fori_loop(0, DP // 16, zcol, 0)
            return carry

        lax.fori_loop(0, K, zrow, 0)

        def zdma(t, carry):
            pltpu.sync_copy(rows_a, acc_sh.at[pl.ds(s * RPS + t * K, K)])
            return carry

        lax.fori_loop(0, RPS // K, zdma, 0)
        _REM = RPS - (RPS // K) * K
        pltpu.sync_copy(rows_a.at[pl.ds(0, _REM)],
                        acc_sh.at[pl.ds(s * RPS + (RPS // K) * K, _REM)])
        plsc.subcore_barrier()

        def group(g, carry):
            base = wid * EPW + g * (CHG * K)
            pltpu.sync_copy(src_hbm.at[pl.ds(base, CHG * K)], src_v)
            pltpu.sync_copy(dst_hbm.at[pl.ds(base, CHG * K)], dst_v)
            pltpu.async_copy(h_hbm.at[src_v.at[pl.ds(0, K)]], rows_a, sem_a)
            pltpu.async_copy(h_hbm.at[src_v.at[pl.ds(K, K)]], rows_b, sem_b)

            def pair(jj, carry2):
                j0 = 2 * jj
                j1 = j0 + 1
                pltpu.make_async_copy(
                    h_hbm.at[src_v.at[pl.ds(j0 * K, K)]], rows_a, sem_a).wait()
                pltpu.sync_copy(rows_a, acc_sh.at[dst_v.at[pl.ds(j0 * K, K)]], add=True)
                pltpu.async_copy(h_hbm.at[src_v.at[pl.ds((j0 + 2) * K, K)]],
                                 rows_a, sem_a)

                pltpu.make_async_copy(
                    h_hbm.at[src_v.at[pl.ds(j1 * K, K)]], rows_b, sem_b).wait()
                pltpu.sync_copy(rows_b, acc_sh.at[dst_v.at[pl.ds(j1 * K, K)]], add=True)

                @pl.when(jj < CHG // 2 - 1)
                def _():
                    pltpu.async_copy(h_hbm.at[src_v.at[pl.ds((j1 + 2) * K, K)]],
                                     rows_b, sem_b)

                return carry2

            lax.fori_loop(0, CHG // 2, pair, 0)
            pltpu.make_async_copy(
                h_hbm.at[src_v.at[pl.ds((CHG - 1) * K, K)]], rows_a, sem_a).wait()
            pltpu.sync_copy(rows_a, acc_sh.at[dst_v.at[pl.ds((CHG - 1) * K, K)]], add=True)
            return carry

        lax.fori_loop(0, G, group, 0)
        plsc.subcore_barrier()
        pltpu.sync_copy(acc_sh.at[pl.ds(s * RPS, RPS)],
                        out_hbm.at[pl.ds(c * NP + s * RPS, RPS)])

    return seg_sum


def _gru_body(p_ref, h_ref, wih_ref, whh_ref, bih_ref, bhh_ref, out_ref):
    x = p_ref[0] + p_ref[1]
    h = h_ref[...]
    gi = jnp.dot(x, wih_ref[...], preferred_element_type=jnp.float32) + bih_ref[...]
    gh = jnp.dot(h, whh_ref[...], preferred_element_type=jnp.float32) + bhh_ref[...]
    i_r, i_z, i_n = gi[:, :DP], gi[:, DP:2 * DP], gi[:, 2 * DP:]
    h_r, h_z, h_n = gh[:, :DP], gh[:, DP:2 * DP], gh[:, 2 * DP:]
    r = jax.nn.sigmoid(i_r + h_r)
    z = jax.nn.sigmoid(i_z + h_z)
    n = jnp.tanh(i_n + r * h_n)
    out_ref[...] = (1.0 - z) * n + z * h


_tc_gru = pl.pallas_call(
    _gru_body,
    grid=(N // RB,),
    in_specs=[
        pl.BlockSpec((NC, RB, DP), lambda i: (0, i, 0)),
        pl.BlockSpec((RB, DP), lambda i: (i, 0)),
        pl.BlockSpec((DP, 3 * DP), lambda i: (0, 0)),
        pl.BlockSpec((DP, 3 * DP), lambda i: (0, 0)),
        pl.BlockSpec((1, 3 * DP), lambda i: (0, 0)),
        pl.BlockSpec((1, 3 * DP), lambda i: (0, 0)),
    ],
    out_specs=pl.BlockSpec((RB, DP), lambda i: (i, 0)),
    out_shape=jax.ShapeDtypeStruct((N, DP), jnp.float32),
)


RB2 = 632
GRID2 = NP // RB2


def _gru_last_body(p0_ref, p1_ref, h_ref, wih_ref, whh_ref, bih_ref, bhh_ref,
                   w1_ref, b1_ref, w2_ref, b2_ref, w3_ref, b3_ref,
                   out_ref, acc_ref):
    i = pl.program_id(0)
    x = p0_ref[...] + p1_ref[...]
    h = h_ref[...]
    gi = jnp.dot(x, wih_ref[...], preferred_element_type=jnp.float32) + bih_ref[...]
    gh = jnp.dot(h, whh_ref[...], preferred_element_type=jnp.float32) + bhh_ref[...]
    i_r, i_z, i_n = gi[:, :DP], gi[:, DP:2 * DP], gi[:, 2 * DP:]
    h_r, h_z, h_n = gh[:, :DP], gh[:, DP:2 * DP], gh[:, 2 * DP:]
    r = jax.nn.sigmoid(i_r + h_r)
    z = jax.nn.sigmoid(i_z + h_z)
    n = jnp.tanh(i_n + r * h_n)
    hn = (1.0 - z) * n + z * h
    rowid = jax.lax.broadcasted_iota(jnp.int32, (RB2, 1), 0) + i * RB2
    hn = jnp.where(rowid < N, hn, 0.0)

    @pl.when(i == 0)
    def _():
        acc_ref[...] = jnp.zeros((1, DP), jnp.float32)

    acc_ref[...] += jnp.sum(hn, axis=0, keepdims=True)

    @pl.when(i == GRID2 - 1)
    def _():
        g = acc_ref[...]
        g = jnp.log(g)
        g = jnp.where(jnp.isnan(g), 0.0, g)
        g = jnp.maximum(g, 0.0)
        y = jnp.dot(g, w1_ref[...], preferred_element_type=jnp.float32) + b1_ref[...]
        y = jnp.where(y >= 0.0, y, 0.01 * y)
        y = jnp.dot(y, w2_ref[...], preferred_element_type=jnp.float32) + b2_ref[...]
        y = jnp.where(y >= 0.0, y, 0.01 * y)
        y = jnp.dot(y, w3_ref[...], preferred_element_type=jnp.float32) + b3_ref[...]
        out_ref[...] = y


_tc_gru_last = pl.pallas_call(
    _gru_last_body,
    grid=(GRID2,),
    in_specs=[
        pl.BlockSpec((RB2, DP), lambda i: (i, 0)),
        pl.BlockSpec((RB2, DP), lambda i: (GRID2 + i, 0)),
        pl.BlockSpec((RB2, DP), lambda i: (i, 0)),
        pl.BlockSpec((DP, 3 * DP), lambda i: (0, 0)),
        pl.BlockSpec((DP, 3 * DP), lambda i: (0, 0)),
        pl.BlockSpec((1, 3 * DP), lambda i: (0, 0)),
        pl.BlockSpec((1, 3 * DP), lambda i: (0, 0)),
        pl.BlockSpec((DP, 80), lambda i: (0, 0)),
        pl.BlockSpec((1, 80), lambda i: (0, 0)),
        pl.BlockSpec((80, 80), lambda i: (0, 0)),
        pl.BlockSpec((1, 80), lambda i: (0, 0)),
        pl.BlockSpec((80, 16), lambda i: (0, 0)),
        pl.BlockSpec((1, 16), lambda i: (0, 0)),
    ],
    out_specs=pl.BlockSpec((1, 16), lambda i: (0, 0)),
    out_shape=jax.ShapeDtypeStruct((1, 16), jnp.float32),
    scratch_shapes=[pltpu.VMEM((1, DP), jnp.float32)],
)

_sc_segment_sum_cache = []


def _sc_segment_sum(h, src, dst):
    if not _sc_segment_sum_cache:
        _sc_segment_sum_cache.append(_build_sc_segment_sum())
    return _sc_segment_sum_cache[0](h, src, dst)


def _pad_gate_weights(w, b):
    w3 = w.reshape(3, D, D)
    wt = jnp.zeros((3, DP, DP), jnp.float32)
    wt = wt.at[:, :D, :D].set(jnp.transpose(w3, (0, 2, 1)))
    wt = jnp.transpose(wt, (1, 0, 2)).reshape(DP, 3 * DP)
    bp = jnp.zeros((3, DP), jnp.float32).at[:, :D].set(b.reshape(3, D)).reshape(1, 3 * DP)
    return wt, bp


def kernel(nodes, edge_index, W_ih, W_hh, b_ih, b_hh,
           fc1_w, fc1_b, fc2_w, fc2_b, fc3_w, fc3_b):
    src = edge_index[0].astype(jnp.int32)
    dst = edge_index[1].astype(jnp.int32)
    h = jnp.zeros((N, DP), jnp.float32).at[:, :D].set(nodes)

    wih, bih = _pad_gate_weights(W_ih, b_ih)
    whh, bhh = _pad_gate_weights(W_hh, b_hh)
    w1 = jnp.zeros((DP, 80), jnp.float32).at[:D, :].set(fc1_w.T)
    b1 = fc1_b.reshape(1, 80)
    w2 = fc2_w.T
    b2 = fc2_b.reshape(1, 80)
    w3 = jnp.zeros((80, 16), jnp.float32).at[:, :10].set(fc3_w.T)
    b3 = jnp.zeros((1, 16), jnp.float32).at[0, :10].set(fc3_b)

    for _ in range(PASSES - 1):
        partials = _sc_segment_sum(h, src, dst).reshape(NC, NP, DP)
        h = _tc_gru(partials, h, wih, whh, bih, bhh)

    pflat = _sc_segment_sum(h, src, dst)
    out = _tc_gru_last(pflat, pflat, h, wih, whh, bih, bhh,
                       w1, b1, w2, b2, w3, b3)
    return out[0, :10]


# speedup vs baseline: 1.0470x; 1.0146x over previous
import functools

import jax
import jax.numpy as jnp
from jax import lax
from jax.experimental import pallas as pl
from jax.experimental.pallas import tpu as pltpu
from jax.experimental.pallas import tpu_sc as plsc

N = 10000
E = 320000
D = 150
DP = 160
PASSES = 4
NC = 2
NS = 16
NW = NC * NS
EPW = E // NW
K = 80
CH = EPW // K
G = 5
CHG = CH // G
NP = 10112
RPS = NP // NS
RB = 1000


def _build_sc_segment_sum():
    mesh = plsc.VectorSubcoreMesh(
        core_axis_name="c", subcore_axis_name="s", num_cores=NC, num_subcores=NS
    )

    @functools.partial(
        pl.kernel,
        out_type=jax.ShapeDtypeStruct((NC * NP, DP), jnp.float32),
        mesh=mesh,
        scratch_types=[
            pltpu.VMEM((CHG * K,), jnp.int32),
            pltpu.VMEM((CHG * K,), jnp.int32),
            pltpu.VMEM((K, DP), jnp.float32),
            pltpu.VMEM((K, DP), jnp.float32),
            pltpu.VMEM_SHARED((NP, DP), jnp.float32),
            pltpu.SemaphoreType.DMA,
            pltpu.SemaphoreType.DMA,
        ],
        compiler_params=pltpu.CompilerParams(use_tc_tiling_on_sc=False),
    )
    def seg_sum(h_hbm, src_hbm, dst_hbm, out_hbm,
                src_v, dst_v, rows_a, rows_b, acc_sh, sem_a, sem_b):
        c = lax.axis_index("c")
        s = lax.axis_index("s")
        wid = s * NC + c
        z16 = jnp.zeros((16,), jnp.float32)

        def zrow(r, carry):
            def zcol(q, carry2):
                rows_a[r, pl.ds(q * 16, 16)] = z16
                return carry2

            lax.fori_loop(0, DP // 16, zcol, 0)
            return carry

        lax.fori_loop(0, K, zrow, 0)

        def zdma(t, carry):
            pltpu.sync_copy(rows_a, acc_sh.at[pl.ds(s * RPS + t * K, K)])
            return carry

        lax.fori_loop(0, RPS // K, zdma, 0)
        _REM = RPS - (RPS // K) * K
        pltpu.sync_copy(rows_a.at[pl.ds(0, _REM)],
                        acc_sh.at[pl.ds(s * RPS + (RPS // K) * K, _REM)])
        plsc.subcore_barrier()

        def group(g, carry):
            base = wid * EPW + g * (CHG * K)
            pltpu.async_copy(src_hbm.at[pl.ds(base, CHG * K)], src_v, sem_a)
            pltpu.async_copy(dst_hbm.at[pl.ds(base, CHG * K)], dst_v, sem_b)
            pltpu.make_async_copy(src_hbm.at[pl.ds(base, CHG * K)], src_v,
                                  sem_a).wait()
            pltpu.make_async_copy(dst_hbm.at[pl.ds(base, CHG * K)], dst_v,
                                  sem_b).wait()
            pltpu.async_copy(h_hbm.at[src_v.at[pl.ds(0, K)]], rows_a, sem_a)
            pltpu.async_copy(h_hbm.at[src_v.at[pl.ds(K, K)]], rows_b, sem_b)

            def pair(jj, carry2):
                j0 = 2 * jj
                j1 = j0 + 1
                pltpu.make_async_copy(
                    h_hbm.at[src_v.at[pl.ds(j0 * K, K)]], rows_a, sem_a).wait()
                pltpu.sync_copy(rows_a, acc_sh.at[dst_v.at[pl.ds(j0 * K, K)]], add=True)
                pltpu.async_copy(h_hbm.at[src_v.at[pl.ds((j0 + 2) * K, K)]],
                                 rows_a, sem_a)

                pltpu.make_async_copy(
                    h_hbm.at[src_v.at[pl.ds(j1 * K, K)]], rows_b, sem_b).wait()
                pltpu.sync_copy(rows_b, acc_sh.at[dst_v.at[pl.ds(j1 * K, K)]], add=True)

                @pl.when(jj < CHG // 2 - 1)
                def _():
                    pltpu.async_copy(h_hbm.at[src_v.at[pl.ds((j1 + 2) * K, K)]],
                                     rows_b, sem_b)

                return carry2

            lax.fori_loop(0, CHG // 2, pair, 0)
            pltpu.make_async_copy(
                h_hbm.at[src_v.at[pl.ds((CHG - 1) * K, K)]], rows_a, sem_a).wait()
            pltpu.sync_copy(rows_a, acc_sh.at[dst_v.at[pl.ds((CHG - 1) * K, K)]], add=True)
            return carry

        lax.fori_loop(0, G, group, 0)
        plsc.subcore_barrier()
        pltpu.sync_copy(acc_sh.at[pl.ds(s * RPS, RPS)],
                        out_hbm.at[pl.ds(c * NP + s * RPS, RPS)])

    return seg_sum


def _gru_body(p_ref, h_ref, wih_ref, whh_ref, bih_ref, bhh_ref, out_ref):
    x = p_ref[0] + p_ref[1]
    h = h_ref[...]
    gi = jnp.dot(x, wih_ref[...], preferred_element_type=jnp.float32) + bih_ref[...]
    gh = jnp.dot(h, whh_ref[...], preferred_element_type=jnp.float32) + bhh_ref[...]
    i_r, i_z, i_n = gi[:, :DP], gi[:, DP:2 * DP], gi[:, 2 * DP:]
    h_r, h_z, h_n = gh[:, :DP], gh[:, DP:2 * DP], gh[:, 2 * DP:]
    r = jax.nn.sigmoid(i_r + h_r)
    z = jax.nn.sigmoid(i_z + h_z)
    n = jnp.tanh(i_n + r * h_n)
    out_ref[...] = (1.0 - z) * n + z * h


_tc_gru = pl.pallas_call(
    _gru_body,
    grid=(N // RB,),
    in_specs=[
        pl.BlockSpec((NC, RB, DP), lambda i: (0, i, 0)),
        pl.BlockSpec((RB, DP), lambda i: (i, 0)),
        pl.BlockSpec((DP, 3 * DP), lambda i: (0, 0)),
        pl.BlockSpec((DP, 3 * DP), lambda i: (0, 0)),
        pl.BlockSpec((1, 3 * DP), lambda i: (0, 0)),
        pl.BlockSpec((1, 3 * DP), lambda i: (0, 0)),
    ],
    out_specs=pl.BlockSpec((RB, DP), lambda i: (i, 0)),
    out_shape=jax.ShapeDtypeStruct((N, DP), jnp.float32),
)


RB2 = 632
GRID2 = NP // RB2


def _gru_last_body(p0_ref, p1_ref, h_ref, wih_ref, whh_ref, bih_ref, bhh_ref,
                   w1_ref, b1_ref, w2_ref, b2_ref, w3_ref, b3_ref,
                   out_ref, acc_ref):
    i = pl.program_id(0)
    x = p0_ref[...] + p1_ref[...]
    h = h_ref[...]
    gi = jnp.dot(x, wih_ref[...], preferred_element_type=jnp.float32) + bih_ref[...]
    gh = jnp.dot(h, whh_ref[...], preferred_element_type=jnp.float32) + bhh_ref[...]
    i_r, i_z, i_n = gi[:, :DP], gi[:, DP:2 * DP], gi[:, 2 * DP:]
    h_r, h_z, h_n = gh[:, :DP], gh[:, DP:2 * DP], gh[:, 2 * DP:]
    r = jax.nn.sigmoid(i_r + h_r)
    z = jax.nn.sigmoid(i_z + h_z)
    n = jnp.tanh(i_n + r * h_n)
    hn = (1.0 - z) * n + z * h
    rowid = jax.lax.broadcasted_iota(jnp.int32, (RB2, 1), 0) + i * RB2
    hn = jnp.where(rowid < N, hn, 0.0)

    @pl.when(i == 0)
    def _():
        acc_ref[...] = jnp.zeros((1, DP), jnp.float32)

    acc_ref[...] += jnp.sum(hn, axis=0, keepdims=True)

    @pl.when(i == GRID2 - 1)
    def _():
        g = acc_ref[...]
        g = jnp.log(g)
        g = jnp.where(jnp.isnan(g), 0.0, g)
        g = jnp.maximum(g, 0.0)
        y = jnp.dot(g, w1_ref[...], preferred_element_type=jnp.float32) + b1_ref[...]
        y = jnp.where(y >= 0.0, y, 0.01 * y)
        y = jnp.dot(y, w2_ref[...], preferred_element_type=jnp.float32) + b2_ref[...]
        y = jnp.where(y >= 0.0, y, 0.01 * y)
        y = jnp.dot(y, w3_ref[...], preferred_element_type=jnp.float32) + b3_ref[...]
        out_ref[...] = y


_tc_gru_last = pl.pallas_call(
    _gru_last_body,
    grid=(GRID2,),
    in_specs=[
        pl.BlockSpec((RB2, DP), lambda i: (i, 0)),
        pl.BlockSpec((RB2, DP), lambda i: (GRID2 + i, 0)),
        pl.BlockSpec((RB2, DP), lambda i: (i, 0)),
        pl.BlockSpec((DP, 3 * DP), lambda i: (0, 0)),
        pl.BlockSpec((DP, 3 * DP), lambda i: (0, 0)),
        pl.BlockSpec((1, 3 * DP), lambda i: (0, 0)),
        pl.BlockSpec((1, 3 * DP), lambda i: (0, 0)),
        pl.BlockSpec((DP, 80), lambda i: (0, 0)),
        pl.BlockSpec((1, 80), lambda i: (0, 0)),
        pl.BlockSpec((80, 80), lambda i: (0, 0)),
        pl.BlockSpec((1, 80), lambda i: (0, 0)),
        pl.BlockSpec((80, 16), lambda i: (0, 0)),
        pl.BlockSpec((1, 16), lambda i: (0, 0)),
    ],
    out_specs=pl.BlockSpec((1, 16), lambda i: (0, 0)),
    out_shape=jax.ShapeDtypeStruct((1, 16), jnp.float32),
    scratch_shapes=[pltpu.VMEM((1, DP), jnp.float32)],
)

_sc_segment_sum_cache = []


def _sc_segment_sum(h, src, dst):
    if not _sc_segment_sum_cache:
        _sc_segment_sum_cache.append(_build_sc_segment_sum())
    return _sc_segment_sum_cache[0](h, src, dst)


def _pad_gate_weights(w, b):
    w3 = w.reshape(3, D, D)
    wt = jnp.zeros((3, DP, DP), jnp.float32)
    wt = wt.at[:, :D, :D].set(jnp.transpose(w3, (0, 2, 1)))
    wt = jnp.transpose(wt, (1, 0, 2)).reshape(DP, 3 * DP)
    bp = jnp.zeros((3, DP), jnp.float32).at[:, :D].set(b.reshape(3, D)).reshape(1, 3 * DP)
    return wt, bp


def kernel(nodes, edge_index, W_ih, W_hh, b_ih, b_hh,
           fc1_w, fc1_b, fc2_w, fc2_b, fc3_w, fc3_b):
    src = edge_index[0].astype(jnp.int32)
    dst = edge_index[1].astype(jnp.int32)
    h = jnp.zeros((N, DP), jnp.float32).at[:, :D].set(nodes)

    wih, bih = _pad_gate_weights(W_ih, b_ih)
    whh, bhh = _pad_gate_weights(W_hh, b_hh)
    w1 = jnp.zeros((DP, 80), jnp.float32).at[:D, :].set(fc1_w.T)
    b1 = fc1_b.reshape(1, 80)
    w2 = fc2_w.T
    b2 = fc2_b.reshape(1, 80)
    w3 = jnp.zeros((80, 16), jnp.float32).at[:, :10].set(fc3_w.T)
    b3 = jnp.zeros((1, 16), jnp.float32).at[0, :10].set(fc3_b)

    for _ in range(PASSES - 1):
        partials = _sc_segment_sum(h, src, dst).reshape(NC, NP, DP)
        h = _tc_gru(partials, h, wih, whh, bih, bhh)

    pflat = _sc_segment_sum(h, src, dst)
    out = _tc_gru_last(pflat, pflat, h, wih, whh, bih, bhh,
                       w1, b1, w2, b2, w3, b3)
    return out[0, :10]


# speedup vs baseline: 1.0538x; 1.0064x over previous
import functools

import jax
import jax.numpy as jnp
from jax import lax
from jax.experimental import pallas as pl
from jax.experimental.pallas import tpu as pltpu
from jax.experimental.pallas import tpu_sc as plsc

N = 10000
E = 320000
D = 150
DP = 160
PASSES = 4
NC = 2
NS = 16
NW = NC * NS
EPW = E // NW
K = 80
CH = EPW // K
G = 5
CHG = CH // G
NP = 10112
RPS = NP // NS
RB = 2000


def _build_sc_segment_sum():
    mesh = plsc.VectorSubcoreMesh(
        core_axis_name="c", subcore_axis_name="s", num_cores=NC, num_subcores=NS
    )

    @functools.partial(
        pl.kernel,
        out_type=jax.ShapeDtypeStruct((NC * NP, DP), jnp.float32),
        mesh=mesh,
        scratch_types=[
            pltpu.VMEM((CHG * K,), jnp.int32),
            pltpu.VMEM((CHG * K,), jnp.int32),
            pltpu.VMEM((K, DP), jnp.float32),
            pltpu.VMEM((K, DP), jnp.float32),
            pltpu.VMEM_SHARED((NP, DP), jnp.float32),
            pltpu.SemaphoreType.DMA,
            pltpu.SemaphoreType.DMA,
        ],
        compiler_params=pltpu.CompilerParams(use_tc_tiling_on_sc=False),
    )
    def seg_sum(h_hbm, src_hbm, dst_hbm, out_hbm,
                src_v, dst_v, rows_a, rows_b, acc_sh, sem_a, sem_b):
        c = lax.axis_index("c")
        s = lax.axis_index("s")
        wid = s * NC + c
        z16 = jnp.zeros((16,), jnp.float32)

        def zrow(r, carry):
            def zcol(q, carry2):
                rows_a[r, pl.ds(q * 16, 16)] = z16
                return carry2

            lax.fori_loop(0, DP // 16, zcol, 0)
            return carry

        lax.fori_loop(0, K, zrow, 0)

        def zdma(t, carry):
            pltpu.sync_copy(rows_a, acc_sh.at[pl.ds(s * RPS + t * K, K)])
            return carry

        lax.fori_loop(0, RPS // K, zdma, 0)
        _REM = RPS - (RPS // K) * K
        pltpu.sync_copy(rows_a.at[pl.ds(0, _REM)],
                        acc_sh.at[pl.ds(s * RPS + (RPS // K) * K, _REM)])
        plsc.subcore_barrier()

        def group(g, carry):
            base = wid * EPW + g * (CHG * K)
            pltpu.async_copy(src_hbm.at[pl.ds(base, CHG * K)], src_v, sem_a)
            pltpu.async_copy(dst_hbm.at[pl.ds(base, CHG * K)], dst_v, sem_b)
            pltpu.make_async_copy(src_hbm.at[pl.ds(base, CHG * K)], src_v,
                                  sem_a).wait()
            pltpu.make_async_copy(dst_hbm.at[pl.ds(base, CHG * K)], dst_v,
                                  sem_b).wait()
            pltpu.async_copy(h_hbm.at[src_v.at[pl.ds(0, K)]], rows_a, sem_a)
            pltpu.async_copy(h_hbm.at[src_v.at[pl.ds(K, K)]], rows_b, sem_b)

            def pair(jj, carry2):
                j0 = 2 * jj
                j1 = j0 + 1
                pltpu.make_async_copy(
                    h_hbm.at[src_v.at[pl.ds(j0 * K, K)]], rows_a, sem_a).wait()
                pltpu.sync_copy(rows_a, acc_sh.at[dst_v.at[pl.ds(j0 * K, K)]], add=True)
                pltpu.async_copy(h_hbm.at[src_v.at[pl.ds((j0 + 2) * K, K)]],
                                 rows_a, sem_a)

                pltpu.make_async_copy(
                    h_hbm.at[src_v.at[pl.ds(j1 * K, K)]], rows_b, sem_b).wait()
                pltpu.sync_copy(rows_b, acc_sh.at[dst_v.at[pl.ds(j1 * K, K)]], add=True)

                @pl.when(jj < CHG // 2 - 1)
                def _():
                    pltpu.async_copy(h_hbm.at[src_v.at[pl.ds((j1 + 2) * K, K)]],
                                     rows_b, sem_b)

                return carry2

            lax.fori_loop(0, CHG // 2, pair, 0)
            pltpu.make_async_copy(
                h_hbm.at[src_v.at[pl.ds((CHG - 1) * K, K)]], rows_a, sem_a).wait()
            pltpu.sync_copy(rows_a, acc_sh.at[dst_v.at[pl.ds((CHG - 1) * K, K)]], add=True)
            return carry

        lax.fori_loop(0, G, group, 0)
        plsc.subcore_barrier()
        pltpu.sync_copy(acc_sh.at[pl.ds(s * RPS, RPS)],
                        out_hbm.at[pl.ds(c * NP + s * RPS, RPS)])

    return seg_sum


def _gru_body(p_ref, h_ref, wih_ref, whh_ref, bih_ref, bhh_ref, out_ref):
    x = p_ref[0] + p_ref[1]
    h = h_ref[...]
    gi = jnp.dot(x, wih_ref[...], preferred_element_type=jnp.float32) + bih_ref[...]
    gh = jnp.dot(h, whh_ref[...], preferred_element_type=jnp.float32) + bhh_ref[...]
    i_r, i_z, i_n = gi[:, :DP], gi[:, DP:2 * DP], gi[:, 2 * DP:]
    h_r, h_z, h_n = gh[:, :DP], gh[:, DP:2 * DP], gh[:, 2 * DP:]
    r = jax.nn.sigmoid(i_r + h_r)
    z = jax.nn.sigmoid(i_z + h_z)
    n = jnp.tanh(i_n + r * h_n)
    out_ref[...] = (1.0 - z) * n + z * h


_tc_gru = pl.pallas_call(
    _gru_body,
    grid=(N // RB,),
    in_specs=[
        pl.BlockSpec((NC, RB, DP), lambda i: (0, i, 0)),
        pl.BlockSpec((RB, DP), lambda i: (i, 0)),
        pl.BlockSpec((DP, 3 * DP), lambda i: (0, 0)),
        pl.BlockSpec((DP, 3 * DP), lambda i: (0, 0)),
        pl.BlockSpec((1, 3 * DP), lambda i: (0, 0)),
        pl.BlockSpec((1, 3 * DP), lambda i: (0, 0)),
    ],
    out_specs=pl.BlockSpec((RB, DP), lambda i: (i, 0)),
    out_shape=jax.ShapeDtypeStruct((N, DP), jnp.float32),
)


RB2 = 632
GRID2 = NP // RB2


def _gru_last_body(p0_ref, p1_ref, h_ref, wih_ref, whh_ref, bih_ref, bhh_ref,
                   w1_ref, b1_ref, w2_ref, b2_ref, w3_ref, b3_ref,
                   out_ref, acc_ref):
    i = pl.program_id(0)
    x = p0_ref[...] + p1_ref[...]
    h = h_ref[...]
    gi = jnp.dot(x, wih_ref[...], preferred_element_type=jnp.float32) + bih_ref[...]
    gh = jnp.dot(h, whh_ref[...], preferred_element_type=jnp.float32) + bhh_ref[...]
    i_r, i_z, i_n = gi[:, :DP], gi[:, DP:2 * DP], gi[:, 2 * DP:]
    h_r, h_z, h_n = gh[:, :DP], gh[:, DP:2 * DP], gh[:, 2 * DP:]
    r = jax.nn.sigmoid(i_r + h_r)
    z = jax.nn.sigmoid(i_z + h_z)
    n = jnp.tanh(i_n + r * h_n)
    hn = (1.0 - z) * n + z * h
    rowid = jax.lax.broadcasted_iota(jnp.int32, (RB2, 1), 0) + i * RB2
    hn = jnp.where(rowid < N, hn, 0.0)

    @pl.when(i == 0)
    def _():
        acc_ref[...] = jnp.zeros((1, DP), jnp.float32)

    acc_ref[...] += jnp.sum(hn, axis=0, keepdims=True)

    @pl.when(i == GRID2 - 1)
    def _():
        g = acc_ref[...]
        g = jnp.log(g)
        g = jnp.where(jnp.isnan(g), 0.0, g)
        g = jnp.maximum(g, 0.0)
        y = jnp.dot(g, w1_ref[...], preferred_element_type=jnp.float32) + b1_ref[...]
        y = jnp.where(y >= 0.0, y, 0.01 * y)
        y = jnp.dot(y, w2_ref[...], preferred_element_type=jnp.float32) + b2_ref[...]
        y = jnp.where(y >= 0.0, y, 0.01 * y)
        y = jnp.dot(y, w3_ref[...], preferred_element_type=jnp.float32) + b3_ref[...]
        out_ref[...] = y


_tc_gru_last = pl.pallas_call(
    _gru_last_body,
    grid=(GRID2,),
    in_specs=[
        pl.BlockSpec((RB2, DP), lambda i: (i, 0)),
        pl.BlockSpec((RB2, DP), lambda i: (GRID2 + i, 0)),
        pl.BlockSpec((RB2, DP), lambda i: (i, 0)),
        pl.BlockSpec((DP, 3 * DP), lambda i: (0, 0)),
        pl.BlockSpec((DP, 3 * DP), lambda i: (0, 0)),
        pl.BlockSpec((1, 3 * DP), lambda i: (0, 0)),
        pl.BlockSpec((1, 3 * DP), lambda i: (0, 0)),
        pl.BlockSpec((DP, 80), lambda i: (0, 0)),
        pl.BlockSpec((1, 80), lambda i: (0, 0)),
        pl.BlockSpec((80, 80), lambda i: (0, 0)),
        pl.BlockSpec((1, 80), lambda i: (0, 0)),
        pl.BlockSpec((80, 16), lambda i: (0, 0)),
        pl.BlockSpec((1, 16), lambda i: (0, 0)),
    ],
    out_specs=pl.BlockSpec((1, 16), lambda i: (0, 0)),
    out_shape=jax.ShapeDtypeStruct((1, 16), jnp.float32),
    scratch_shapes=[pltpu.VMEM((1, DP), jnp.float32)],
)

_sc_segment_sum_cache = []


def _sc_segment_sum(h, src, dst):
    if not _sc_segment_sum_cache:
        _sc_segment_sum_cache.append(_build_sc_segment_sum())
    return _sc_segment_sum_cache[0](h, src, dst)


def _pad_gate_weights(w, b):
    w3 = w.reshape(3, D, D)
    wt = jnp.zeros((3, DP, DP), jnp.float32)
    wt = wt.at[:, :D, :D].set(jnp.transpose(w3, (0, 2, 1)))
    wt = jnp.transpose(wt, (1, 0, 2)).reshape(DP, 3 * DP)
    bp = jnp.zeros((3, DP), jnp.float32).at[:, :D].set(b.reshape(3, D)).reshape(1, 3 * DP)
    return wt, bp


def kernel(nodes, edge_index, W_ih, W_hh, b_ih, b_hh,
           fc1_w, fc1_b, fc2_w, fc2_b, fc3_w, fc3_b):
    src = edge_index[0].astype(jnp.int32)
    dst = edge_index[1].astype(jnp.int32)
    h = jnp.zeros((N, DP), jnp.float32).at[:, :D].set(nodes)

    wih, bih = _pad_gate_weights(W_ih, b_ih)
    whh, bhh = _pad_gate_weights(W_hh, b_hh)
    w1 = jnp.zeros((DP, 80), jnp.float32).at[:D, :].set(fc1_w.T)
    b1 = fc1_b.reshape(1, 80)
    w2 = fc2_w.T
    b2 = fc2_b.reshape(1, 80)
    w3 = jnp.zeros((80, 16), jnp.float32).at[:, :10].set(fc3_w.T)
    b3 = jnp.zeros((1, 16), jnp.float32).at[0, :10].set(fc3_b)

    for _ in range(PASSES - 1):
        partials = _sc_segment_sum(h, src, dst).reshape(NC, NP, DP)
        h = _tc_gru(partials, h, wih, whh, bih, bhh)

    pflat = _sc_segment_sum(h, src, dst)
    out = _tc_gru_last(pflat, pflat, h, wih, whh, bih, bhh,
                       w1, b1, w2, b2, w3, b3)
    return out[0, :10]
